# Initial kernel scaffold; baseline (speedup 1.0000x reference)
import functools

import jax
import jax.numpy as jnp
from jax import lax
from jax.experimental import pallas as pl
from jax.experimental.pallas import tpu as pltpu
from jax.experimental.pallas import tpu_sc as plsc

D = 32
NC = 2
NS = 16
NW = NC * NS
CHUNK = 128
NBUF = 4


def _make_kernel(total):
    per_w = total // NW
    nchunk = per_w // CHUNK

    mesh = plsc.VectorSubcoreMesh(core_axis_name="c", subcore_axis_name="s")

    @functools.partial(
        pl.kernel,
        mesh=mesh,
        out_type=jax.ShapeDtypeStruct((total, D), jnp.float32),
        scratch_types=[
            pltpu.VMEM((nchunk, CHUNK), jnp.int32),
            pltpu.VMEM((NBUF, CHUNK, D), jnp.float32),
            pltpu.SemaphoreType.DMA((NBUF,)),
            pltpu.SemaphoreType.DMA((NBUF,)),
        ],
    )
    def k(table_hbm, idx_hbm, out_hbm, idx_v, rows_v, gsem, ssem):
        wid = lax.axis_index("s") * NC + lax.axis_index("c")
        base = wid * per_w
        pltpu.sync_copy(idx_hbm.at[wid], idx_v)

        def gather(j, b):
            return pltpu.make_async_copy(
                table_hbm.at[idx_v.at[j]], rows_v.at[b], gsem.at[b]
            )

        def store(j, b):
            return pltpu.make_async_copy(
                rows_v.at[b],
                out_hbm.at[pl.ds(base + j * CHUNK, CHUNK)],
                ssem.at[b],
            )

        for b in range(NBUF):
            gather(b, b).start()

        @pl.loop(0, nchunk, step=NBUF)
        def _(j0):
            for b in range(NBUF):
                j = j0 + b
                gather(j, b).wait()
                store(j, b).start()
            for b in range(NBUF):
                j = j0 + b
                jn = j + NBUF

                @pl.when(jn < nchunk)
                def _():
                    store(j, b).wait()
                    gather(jn, b).start()

        for b in range(NBUF):
            store(nchunk - NBUF + b, b).wait()

    return k


def kernel(X, position_table):
    batch, hist = X.shape
    total = batch * hist
    idx = X.astype(jnp.int32).reshape(NW, total // (NW * CHUNK), CHUNK)
    out = _make_kernel(total)(position_table, idx)
    return out.reshape(batch, hist, D)


# speedup vs baseline: 1.4826x; 1.4826x over previous
import functools

import jax
import jax.numpy as jnp
from jax import lax
from jax.experimental import pallas as pl
from jax.experimental.pallas import tpu as pltpu
from jax.experimental.pallas import tpu_sc as plsc

D = 32
NC = 2
NS = 16
NW = NC * NS
CHUNK = 128
NBUF = 4


def _make_kernel(total):
    per_w = total // NW
    nchunk = per_w // CHUNK

    mesh = plsc.VectorSubcoreMesh(core_axis_name="c", subcore_axis_name="s")

    @functools.partial(
        pl.kernel,
        mesh=mesh,
        out_type=jax.ShapeDtypeStruct((total, D), jnp.float32),
        compiler_params=pltpu.CompilerParams(use_tc_tiling_on_sc=False),
        scratch_types=[
            pltpu.VMEM((nchunk, CHUNK), jnp.int32),
            pltpu.VMEM((NBUF, CHUNK, D), jnp.float32),
            pltpu.SemaphoreType.DMA((NBUF,)),
            pltpu.SemaphoreType.DMA((NBUF,)),
        ],
    )
    def k(table_hbm, idx_hbm, out_hbm, idx_v, rows_v, gsem, ssem):
        wid = lax.axis_index("s") * NC + lax.axis_index("c")
        base = wid * per_w
        pltpu.sync_copy(idx_hbm.at[wid], idx_v)

        def gather(j, b):
            return pltpu.make_async_copy(
                table_hbm.at[idx_v.at[j]], rows_v.at[b], gsem.at[b]
            )

        def store(j, b):
            return pltpu.make_async_copy(
                rows_v.at[b],
                out_hbm.at[pl.ds(base + j * CHUNK, CHUNK)],
                ssem.at[b],
            )

        for b in range(NBUF):
            gather(b, b).start()

        @pl.loop(0, nchunk, step=NBUF)
        def _(j0):
            for b in range(NBUF):
                j = j0 + b
                gather(j, b).wait()
                store(j, b).start()
            for b in range(NBUF):
                j = j0 + b
                jn = j + NBUF

                @pl.when(jn < nchunk)
                def _():
                    store(j, b).wait()
                    gather(jn, b).start()

        for b in range(NBUF):
            store(nchunk - NBUF + b, b).wait()

    return k


def kernel(X, position_table):
    batch, hist = X.shape
    total = batch * hist
    idx = X.astype(jnp.int32).reshape(NW, total // (NW * CHUNK), CHUNK)
    out = _make_kernel(total)(position_table, idx)
    return out.reshape(batch, hist, D)


# speedup vs baseline: 1.4994x; 1.0113x over previous
import functools

import jax
import jax.numpy as jnp
from jax import lax
from jax.experimental import pallas as pl
from jax.experimental.pallas import tpu as pltpu
from jax.experimental.pallas import tpu_sc as plsc

D = 32
NC = 2
NS = 16
NW = NC * NS
CHUNK = 128
NBUF = 8


def _make_kernel(total):
    per_w = total // NW
    nchunk = per_w // CHUNK

    mesh = plsc.VectorSubcoreMesh(core_axis_name="c", subcore_axis_name="s")

    @functools.partial(
        pl.kernel,
        mesh=mesh,
        out_type=jax.ShapeDtypeStruct((total, D), jnp.float32),
        compiler_params=pltpu.CompilerParams(use_tc_tiling_on_sc=False),
        scratch_types=[
            pltpu.VMEM((nchunk, CHUNK), jnp.int32),
            pltpu.VMEM((NBUF, CHUNK, D), jnp.float32),
            pltpu.SemaphoreType.DMA((NBUF,)),
            pltpu.SemaphoreType.DMA((NBUF,)),
        ],
    )
    def k(table_hbm, idx_hbm, out_hbm, idx_v, rows_v, gsem, ssem):
        wid = lax.axis_index("s") * NC + lax.axis_index("c")
        base = wid * per_w
        pltpu.sync_copy(idx_hbm.at[wid], idx_v)

        def gather(j, b):
            return pltpu.make_async_copy(
                table_hbm.at[idx_v.at[j]], rows_v.at[b], gsem.at[b]
            )

        def store(j, b):
            return pltpu.make_async_copy(
                rows_v.at[b],
                out_hbm.at[pl.ds(base + j * CHUNK, CHUNK)],
                ssem.at[b],
            )

        for b in range(NBUF):
            gather(b, b).start()

        @pl.loop(0, nchunk, step=NBUF)
        def _(j0):
            for b in range(NBUF):
                j = j0 + b
                gather(j, b).wait()
                store(j, b).start()
            for b in range(NBUF):
                j = j0 + b
                jn = j + NBUF

                @pl.when(jn < nchunk)
                def _():
                    store(j, b).wait()
                    gather(jn, b).start()

        for b in range(NBUF):
            store(nchunk - NBUF + b, b).wait()

    return k


def kernel(X, position_table):
    batch, hist = X.shape
    total = batch * hist
    idx = X.astype(jnp.int32).reshape(NW, total // (NW * CHUNK), CHUNK)
    out = _make_kernel(total)(position_table, idx)
    return out.reshape(batch, hist, D)
